# baseline (speedup 1.0000x reference)
import jax
import jax.numpy as jnp
from jax import lax
from jax.experimental import pallas as pl
from jax.experimental.pallas import tpu as pltpu
from jax.experimental.pallas import tpu_sc as plsc

_NC = 2
_NS = 16
_L = 16
_NW = _NC * _NS
_B = 16384
_E = 1000000
_D = 32
_BPW = _B // _NW
_CH = 128
_NCH = _BPW // _CH
_G = _CH // _L


def _trig_body(r_ref, c_ref, s_ref):
    c_ref[...] = jnp.cos(r_ref[...])
    s_ref[...] = jnp.sin(r_ref[...])


def _trig_tables(rel):
    cos_t, sin_t = pl.pallas_call(
        _trig_body,
        out_shape=(
            jax.ShapeDtypeStruct(rel.shape, rel.dtype),
            jax.ShapeDtypeStruct(rel.shape, rel.dtype),
        ),
    )(rel)
    return jnp.concatenate([cos_t, sin_t], axis=1).reshape(500, 128)


def _pad_body(x_ref, o_ref):
    o_ref[...] = jnp.concatenate([x_ref[...], jnp.zeros_like(x_ref[...])], axis=1)


def _pad_tc(ent):
    rows = 4096
    return pl.pallas_call(
        _pad_body,
        grid=(_E // rows,),
        in_specs=[pl.BlockSpec((rows, 2 * _D), lambda i: (i, 0))],
        out_specs=pl.BlockSpec((rows, 4 * _D), lambda i: (i, 0)),
        out_shape=jax.ShapeDtypeStruct((_E, 4 * _D), jnp.float32),
    )(ent)


def _score_body(hidx_hbm, ridx_hbm, tidx_hbm, ent_hbm, cs_hbm, out_hbm,
                hidx_v, tidx_v, ridx_v, rsh_v,
                hrows, trows, csrows, out_v, sem, rsem):
    wid = lax.axis_index("s") * _NC + lax.axis_index("c")
    row0 = wid * _NCH
    pltpu.sync_copy(hidx_hbm.at[pl.ds(row0, _NCH)], hidx_v)
    pltpu.sync_copy(tidx_hbm.at[pl.ds(row0, _NCH)], tidx_v)
    pltpu.sync_copy(ridx_hbm.at[pl.ds(row0, _NCH)], ridx_v)
    for k in range(_NCH):
        for g in range(_G):
            sl = pl.ds(g * _L, _L)
            rsh_v[k, sl] = lax.shift_right_logical(ridx_v[k, sl], 1)

    lane = lax.iota(jnp.int32, _L)
    for k in range(_NCH):
        copies = [
            pltpu.async_copy(ent_hbm.at[hidx_v.at[k]], hrows, sem),
            pltpu.async_copy(ent_hbm.at[tidx_v.at[k]], trows, sem),
            pltpu.async_copy(cs_hbm.at[rsh_v.at[k]], csrows, rsem),
        ]
        for cp in copies:
            cp.wait()

        for g in range(_G):
            sl = pl.ds(g * _L, _L)
            rows = g * _L + lane
            pr = (ridx_v[k, sl] & 1) * 64

            def dbody(d, acc, rows=rows, pr=pr):
                dd = jnp.zeros((_L,), jnp.int32) + d
                re = plsc.load_gather(hrows, [rows, dd])
                im = plsc.load_gather(hrows, [rows, dd + _D])
                tre = plsc.load_gather(trows, [rows, dd])
                tim = plsc.load_gather(trows, [rows, dd + _D])
                c = plsc.load_gather(csrows, [rows, pr + d])
                s = plsc.load_gather(csrows, [rows, pr + d + _D])
                return (acc + jnp.abs(re * c - im * s - tre)
                        + jnp.abs(re * s + im * c - tim))

            acc = lax.fori_loop(0, _D, dbody, jnp.zeros((_L,), jnp.float32))
            out_v[pl.ds(k * _CH + g * _L, _L)] = -acc
    pltpu.sync_copy(out_v, out_hbm.at[pl.ds(wid * _BPW, _BPW)])


def _score(hidx2d, ridx2d, tidx2d, ent_pad, cs):
    mesh = plsc.VectorSubcoreMesh(
        core_axis_name="c", subcore_axis_name="s",
        num_cores=_NC, num_subcores=_NS,
    )
    return pl.kernel(
        _score_body,
        out_type=jax.ShapeDtypeStruct((_B,), jnp.float32),
        mesh=mesh,
        compiler_params=pltpu.CompilerParams(
            needs_layout_passes=False, use_tc_tiling_on_sc=False),
        scratch_types=[
            pltpu.VMEM((_NCH, _CH), jnp.int32),
            pltpu.VMEM((_NCH, _CH), jnp.int32),
            pltpu.VMEM((_NCH, _CH), jnp.int32),
            pltpu.VMEM((_NCH, _CH), jnp.int32),
            pltpu.VMEM((_CH, 128), jnp.float32),
            pltpu.VMEM((_CH, 128), jnp.float32),
            pltpu.VMEM((_CH, 128), jnp.float32),
            pltpu.VMEM((_BPW,), jnp.float32),
            pltpu.SemaphoreType.DMA,
            pltpu.SemaphoreType.DMA,
        ],
    )(hidx2d, ridx2d, tidx2d, ent_pad, cs)


def kernel(head_idx, relation_idx, tail_idx, entity_embeddings, relation_embeddings):
    cs = _trig_tables(relation_embeddings)
    ent_pad = _pad_tc(entity_embeddings)
    h2 = head_idx.reshape(_NW * _NCH, _CH)
    r2 = relation_idx.reshape(_NW * _NCH, _CH)
    t2 = tail_idx.reshape(_NW * _NCH, _CH)
    return _score(h2, r2, t2, ent_pad, cs)


# speedup vs baseline: 1.2535x; 1.2535x over previous
import jax
import jax.numpy as jnp
from jax import lax
from jax.experimental import pallas as pl
from jax.experimental.pallas import tpu as pltpu
from jax.experimental.pallas import tpu_sc as plsc

_NC = 2
_NS = 16
_L = 16
_NW = _NC * _NS
_B = 16384
_E = 1000000
_D = 32
_BPW = _B // _NW
_CH = 128
_NCH = _BPW // _CH
_G = _CH // _L


def _trig_body(r_ref, c_ref, s_ref):
    c_ref[...] = jnp.cos(r_ref[...])
    s_ref[...] = jnp.sin(r_ref[...])


def _trig_tables(rel):
    cos_t, sin_t = pl.pallas_call(
        _trig_body,
        out_shape=(
            jax.ShapeDtypeStruct(rel.shape, rel.dtype),
            jax.ShapeDtypeStruct(rel.shape, rel.dtype),
        ),
    )(rel)
    return jnp.concatenate([cos_t, sin_t], axis=1).reshape(500, 128)


def _score_body(hidx_hbm, ridx_hbm, tidx_hbm, ent_hbm, cs_hbm, out_hbm,
                hidx_v, tidx_v, ridx_v, rsh_v,
                hrows, trows, csrows, out_v, sem, rsem):
    wid = lax.axis_index("s") * _NC + lax.axis_index("c")
    row0 = wid * _NCH
    pltpu.sync_copy(hidx_hbm.at[pl.ds(row0, _NCH)], hidx_v)
    pltpu.sync_copy(tidx_hbm.at[pl.ds(row0, _NCH)], tidx_v)
    pltpu.sync_copy(ridx_hbm.at[pl.ds(row0, _NCH)], ridx_v)
    for k in range(_NCH):
        for g in range(_G):
            sl = pl.ds(g * _L, _L)
            rsh_v[k, sl] = lax.shift_right_logical(ridx_v[k, sl], 1)

    lane = lax.iota(jnp.int32, _L)
    for k in range(_NCH):
        copies = [
            pltpu.async_copy(ent_hbm.at[hidx_v.at[k]], hrows, sem),
            pltpu.async_copy(ent_hbm.at[tidx_v.at[k]], trows, sem),
            pltpu.async_copy(cs_hbm.at[rsh_v.at[k]], csrows, rsem),
        ]
        for cp in copies:
            cp.wait()

        for g in range(_G):
            sl = pl.ds(g * _L, _L)
            rows = g * _L + lane
            pr = (ridx_v[k, sl] & 1) * 64

            def dbody(d, acc, rows=rows, pr=pr):
                dd = jnp.zeros((_L,), jnp.int32) + d
                re = plsc.load_gather(hrows, [rows, dd])
                im = plsc.load_gather(hrows, [rows, dd + _D])
                tre = plsc.load_gather(trows, [rows, dd])
                tim = plsc.load_gather(trows, [rows, dd + _D])
                c = plsc.load_gather(csrows, [rows, pr + d])
                s = plsc.load_gather(csrows, [rows, pr + d + _D])
                return (acc + jnp.abs(re * c - im * s - tre)
                        + jnp.abs(re * s + im * c - tim))

            acc = lax.fori_loop(0, _D, dbody, jnp.zeros((_L,), jnp.float32))
            out_v[pl.ds(k * _CH + g * _L, _L)] = -acc
    pltpu.sync_copy(out_v, out_hbm.at[pl.ds(wid * _BPW, _BPW)])


def _score(hidx2d, ridx2d, tidx2d, ent_pad, cs):
    mesh = plsc.VectorSubcoreMesh(
        core_axis_name="c", subcore_axis_name="s",
        num_cores=_NC, num_subcores=_NS,
    )
    return pl.kernel(
        _score_body,
        out_type=jax.ShapeDtypeStruct((_B,), jnp.float32),
        mesh=mesh,
        compiler_params=pltpu.CompilerParams(
            needs_layout_passes=False, use_tc_tiling_on_sc=False),
        scratch_types=[
            pltpu.VMEM((_NCH, _CH), jnp.int32),
            pltpu.VMEM((_NCH, _CH), jnp.int32),
            pltpu.VMEM((_NCH, _CH), jnp.int32),
            pltpu.VMEM((_NCH, _CH), jnp.int32),
            pltpu.VMEM((_CH, 128), jnp.float32),
            pltpu.VMEM((_CH, 128), jnp.float32),
            pltpu.VMEM((_CH, 128), jnp.float32),
            pltpu.VMEM((_BPW,), jnp.float32),
            pltpu.SemaphoreType.DMA,
            pltpu.SemaphoreType.DMA,
        ],
    )(hidx2d, ridx2d, tidx2d, ent_pad, cs)


def kernel(head_idx, relation_idx, tail_idx, entity_embeddings, relation_embeddings):
    cs = _trig_tables(relation_embeddings)
    ent_pad = jnp.concatenate(
        [entity_embeddings, jnp.zeros_like(entity_embeddings)], axis=1)
    h2 = head_idx.reshape(_NW * _NCH, _CH)
    r2 = relation_idx.reshape(_NW * _NCH, _CH)
    t2 = tail_idx.reshape(_NW * _NCH, _CH)
    return _score(h2, r2, t2, ent_pad, cs)
